# baseline (speedup 1.0000x reference)
import functools

import jax
import jax.numpy as jnp
from jax import lax
from jax.experimental import pallas as pl
from jax.experimental.pallas import tpu as pltpu
from jax.experimental.pallas import tpu_sc as plsc

N = 10000
E = 320000
D = 128
G = 16

NC = 2
NS = 16
NW = NC * NS
CHUNK = 128
NROWSP = 2560
EPAD = NROWSP * CHUNK - E
CPT = NROWSP // NW
NACC = 10112
RPT = NACC // NS
HCPT = CPT // 2
CPT0 = 96
CPT1 = 64
SBUF = CPT0 // 2
DBUF = CPT0 // 4

DEG_W = 16
NPADD = 10240
DEG_RPT = NPADD // NS

_mesh = plsc.VectorSubcoreMesh(core_axis_name="c", subcore_axis_name="s")


@functools.partial(
    pl.kernel,
    out_type=jax.ShapeDtypeStruct((NC, NACC, D), jnp.float32),
    mesh=_mesh,
    scratch_types=[
        pltpu.VMEM((CPT, CHUNK), jnp.int32),
        pltpu.VMEM((CHUNK, D), jnp.float32),
        pltpu.VMEM_SHARED((NACC, D), jnp.float32),
        pltpu.SemaphoreType.DMA,
    ],
)
def _sc_deg(dstr_hbm, ones_hbm, zeros_hbm, out_hbm, dst_v, ones_v, acc_sh,
            sem):
    c = lax.axis_index("c")
    s = lax.axis_index("s")
    wid = s * NC + c

    pltpu.sync_copy(zeros_hbm.at[pl.ds(s * RPT, RPT)],
                    acc_sh.at[pl.ds(s * RPT, RPT)])
    pltpu.sync_copy(ones_hbm, ones_v)
    pltpu.sync_copy(dstr_hbm.at[pl.ds(wid * CPT, CPT)], dst_v)

    plsc.subcore_barrier()

    K8 = 8

    def step(j8, carry):
        base = j8 * K8
        for k in range(K8):
            pltpu.async_copy(ones_v, acc_sh.at[dst_v.at[base + k]], sem,
                             add=True)
        for k in range(K8):
            pltpu.make_async_copy(ones_v, acc_sh.at[dst_v.at[base + k]],
                                  sem).wait()
        return carry

    lax.fori_loop(0, CPT // K8, step, 0)

    plsc.subcore_barrier()
    pltpu.sync_copy(acc_sh.at[pl.ds(s * RPT, RPT)],
                    out_hbm.at[c].at[pl.ds(s * RPT, RPT)])


@functools.partial(
    pl.kernel,
    out_type=jax.ShapeDtypeStruct((NC, NACC, D), jnp.float32),
    mesh=_mesh,
    scratch_types=[
        pltpu.VMEM((SBUF, CHUNK), jnp.int32),
        pltpu.VMEM((DBUF, CHUNK), jnp.int32),
        pltpu.VMEM((CHUNK, D), jnp.float32),
        pltpu.VMEM((CHUNK, D), jnp.float32),
        pltpu.VMEM_SHARED((NACC, D), jnp.float32),
        pltpu.SemaphoreType.DMA,
        pltpu.SemaphoreType.DMA,
        pltpu.SemaphoreType.DMA,
        pltpu.SemaphoreType.DMA,
    ],
)
def _sc_agg(u_hbm, srcr_hbm, dstr_hbm, zeros_hbm, out_hbm,
            src_v, dst_v, gbuf0, gbuf1, acc_sh, sem0, sem1, sem2, sem3):
    c = lax.axis_index("c")
    s = lax.axis_index("s")

    pltpu.sync_copy(zeros_hbm.at[pl.ds(s * RPT, RPT)],
                    acc_sh.at[pl.ds(s * RPT, RPT)])

    def run_core(cpt, base):
        sh = cpt // 2
        dq = cpt // 4
        np2 = cpt // 2

        pltpu.sync_copy(srcr_hbm.at[pl.ds(base, sh)], src_v.at[pl.ds(0, sh)])
        pltpu.sync_copy(dstr_hbm.at[pl.ds(base, dq)], dst_v.at[pl.ds(0, dq)])

        pltpu.async_copy(u_hbm.at[src_v.at[0]], gbuf0, sem0)
        pltpu.async_copy(u_hbm.at[src_v.at[1]], gbuf1, sem1)

        def step(j2, carry):
            c0 = 2 * j2
            seg_d = c0 // dq
            r0 = c0 - seg_d * dq
            rs2 = (c0 + 2) - ((c0 + 2) // sh) * sh
            rs3 = (c0 + 3) - ((c0 + 3) // sh) * sh

            @pl.when((c0 > 0) & (r0 == 0))
            def _():
                pltpu.sync_copy(dstr_hbm.at[pl.ds(base + seg_d * dq, dq)],
                                dst_v.at[pl.ds(0, dq)])

            pltpu.make_async_copy(u_hbm.at[src_v.at[0]], gbuf0, sem0).wait()
            pltpu.async_copy(gbuf0, acc_sh.at[dst_v.at[r0]], sem2, add=True)
            pltpu.make_async_copy(u_hbm.at[src_v.at[0]], gbuf1, sem1).wait()
            pltpu.async_copy(gbuf1, acc_sh.at[dst_v.at[r0 + 1]], sem3,
                             add=True)

            @pl.when(c0 + 2 == sh)
            def _():
                pltpu.sync_copy(srcr_hbm.at[pl.ds(base + sh, sh)],
                                src_v.at[pl.ds(0, sh)])

            @pl.when(j2 + 1 < np2)
            def _():
                pltpu.make_async_copy(gbuf0, acc_sh.at[dst_v.at[r0]],
                                      sem2).wait()
                pltpu.async_copy(u_hbm.at[src_v.at[rs2]], gbuf0, sem0)
                pltpu.make_async_copy(gbuf1, acc_sh.at[dst_v.at[r0 + 1]],
                                      sem3).wait()
                pltpu.async_copy(u_hbm.at[src_v.at[rs3]], gbuf1, sem1)
            return carry

        lax.fori_loop(0, np2, step, 0)

        pltpu.make_async_copy(gbuf0, acc_sh.at[dst_v.at[0]], sem2).wait()
        pltpu.make_async_copy(gbuf1, acc_sh.at[dst_v.at[0]], sem3).wait()

    plsc.subcore_barrier()

    @pl.when(c == 0)
    def _():
        run_core(CPT0, s * CPT0)

    @pl.when(c == 1)
    def _():
        run_core(CPT1, NS * CPT0 + s * CPT1)

    plsc.subcore_barrier()
    pltpu.sync_copy(acc_sh.at[pl.ds(s * RPT, RPT)],
                    out_hbm.at[c].at[pl.ds(s * RPT, RPT)])


BN = 1000
GRID = N // BN


def _mm1_body(x_ref, w_ref, dega_ref, degb_ref, o_ref):
    dinv = lax.rsqrt(dega_ref[...] + degb_ref[...] + 1.0)
    h = jnp.dot(x_ref[...], w_ref[...], preferred_element_type=jnp.float32)
    o_ref[...] = dinv * h


_tc_mm1 = pl.pallas_call(
    _mm1_body,
    grid=(GRID,),
    in_specs=[
        pl.BlockSpec((BN, D), lambda i: (i, 0)),
        pl.BlockSpec((D, D), lambda i: (0, 0)),
        pl.BlockSpec((BN, 1), lambda i: (i, 0)),
        pl.BlockSpec((BN, 1), lambda i: (i, 0)),
    ],
    out_specs=pl.BlockSpec((BN, D), lambda i: (i, 0)),
    out_shape=jax.ShapeDtypeStruct((N, D), jnp.float32),
)


def _mm2_body(acca_ref, accb_ref, u_ref, dega_ref, degb_ref, b_ref, w_ref,
              o_ref):
    dinv = lax.rsqrt(dega_ref[...] + degb_ref[...] + 1.0)
    z = dinv * (acca_ref[...] + accb_ref[...] + u_ref[...]) + b_ref[...]
    z = jnp.maximum(z, 0.0)
    h = jnp.dot(z, w_ref[...], preferred_element_type=jnp.float32)
    o_ref[...] = dinv * h


_tc_mm2 = pl.pallas_call(
    _mm2_body,
    grid=(GRID,),
    in_specs=[
        pl.BlockSpec((BN, D), lambda i: (i, 0)),
        pl.BlockSpec((BN, D), lambda i: (i, 0)),
        pl.BlockSpec((BN, D), lambda i: (i, 0)),
        pl.BlockSpec((BN, 1), lambda i: (i, 0)),
        pl.BlockSpec((BN, 1), lambda i: (i, 0)),
        pl.BlockSpec((1, D), lambda i: (0, 0)),
        pl.BlockSpec((D, D), lambda i: (0, 0)),
    ],
    out_specs=pl.BlockSpec((BN, D), lambda i: (i, 0)),
    out_shape=jax.ShapeDtypeStruct((N, D), jnp.float32),
)


def _pool_body(acca_ref, accb_ref, u_ref, dega_ref, degb_ref, b_ref,
               batch_ref, o_ref, sums_ref, counts_ref):
    i = pl.program_id(0)

    @pl.when(i == 0)
    def _():
        sums_ref[...] = jnp.zeros_like(sums_ref)
        counts_ref[...] = jnp.zeros_like(counts_ref)

    dinv = lax.rsqrt(dega_ref[...] + degb_ref[...] + 1.0)
    h = dinv * (acca_ref[...] + accb_ref[...] + u_ref[...]) + b_ref[...]
    gids = lax.broadcasted_iota(jnp.int32, (1, G), 1).astype(jnp.float32)
    mask = (batch_ref[...] == gids).astype(jnp.float32)
    sums_ref[...] += lax.dot_general(
        mask, h, (((0,), (0,)), ((), ())),
        preferred_element_type=jnp.float32)
    counts_ref[...] += jnp.sum(mask, axis=0)[:, None]

    @pl.when(i == GRID - 1)
    def _():
        o_ref[...] = sums_ref[...] / jnp.maximum(counts_ref[...], 1.0)


_tc_pool = pl.pallas_call(
    _pool_body,
    grid=(GRID,),
    in_specs=[
        pl.BlockSpec((BN, D), lambda i: (i, 0)),
        pl.BlockSpec((BN, D), lambda i: (i, 0)),
        pl.BlockSpec((BN, D), lambda i: (i, 0)),
        pl.BlockSpec((BN, 1), lambda i: (i, 0)),
        pl.BlockSpec((BN, 1), lambda i: (i, 0)),
        pl.BlockSpec((1, D), lambda i: (0, 0)),
        pl.BlockSpec((BN, 1), lambda i: (i, 0)),
    ],
    out_specs=pl.BlockSpec((G, D), lambda i: (0, 0)),
    out_shape=jax.ShapeDtypeStruct((G, D), jnp.float32),
    scratch_shapes=[
        pltpu.VMEM((G, D), jnp.float32),
        pltpu.VMEM((G, 1), jnp.float32),
    ],
)


def kernel(x, edge_index, batch_index, W1, b1, W2, b2):
    pad_src = jnp.zeros((EPAD,), jnp.int32)
    pad_dst = jnp.full((EPAD,), N, jnp.int32)
    srcr = jnp.concatenate([edge_index[0], pad_src]).reshape(NROWSP, CHUNK)
    dstr = jnp.concatenate([edge_index[1], pad_dst]).reshape(NROWSP, CHUNK)
    batch_col = batch_index.astype(jnp.float32).reshape(N, 1)

    zeros_agg = jnp.zeros((NACC, D), jnp.float32)
    ones_nd = jnp.ones((CHUNK, D), jnp.float32)

    degp = _sc_deg(dstr, ones_nd, zeros_agg)
    dega = degp[0, :N, 0:1]
    degb = degp[1, :N, 0:1]

    u1 = _tc_mm1(x, W1, dega, degb)
    acc1 = _sc_agg(u1, srcr, dstr, zeros_agg)
    u2 = _tc_mm2(acc1[0, :N], acc1[1, :N], u1, dega, degb,
                 b1.reshape(1, D), W2)
    acc2 = _sc_agg(u2, srcr, dstr, zeros_agg)
    return _tc_pool(acc2[0, :N], acc2[1, :N], u2, dega, degb,
                    b2.reshape(1, D), batch_col)


# speedup vs baseline: 1.0518x; 1.0518x over previous
import functools

import jax
import jax.numpy as jnp
from jax import lax
from jax.experimental import pallas as pl
from jax.experimental.pallas import tpu as pltpu
from jax.experimental.pallas import tpu_sc as plsc

N = 10000
E = 320000
D = 128
G = 16

NC = 2
NS = 16
NW = NC * NS
CHUNK = 128
NROWSP = 2560
EPAD = NROWSP * CHUNK - E
CPT = NROWSP // NW
NACC = 10112
RPT = NACC // NS
HCPT = CPT // 2
CPT0 = 128
CPT1 = 32
SBUF = CPT0 // 2
DBUF = CPT0 // 4

DEG_W = 16
NPADD = 10240
DEG_RPT = NPADD // NS

_mesh = plsc.VectorSubcoreMesh(core_axis_name="c", subcore_axis_name="s")


@functools.partial(
    pl.kernel,
    out_type=jax.ShapeDtypeStruct((NC, NACC, D), jnp.float32),
    mesh=_mesh,
    scratch_types=[
        pltpu.VMEM((CPT, CHUNK), jnp.int32),
        pltpu.VMEM((CHUNK, D), jnp.float32),
        pltpu.VMEM_SHARED((NACC, D), jnp.float32),
        pltpu.SemaphoreType.DMA,
    ],
)
def _sc_deg(dstr_hbm, ones_hbm, zeros_hbm, out_hbm, dst_v, ones_v, acc_sh,
            sem):
    c = lax.axis_index("c")
    s = lax.axis_index("s")
    wid = s * NC + c

    pltpu.sync_copy(zeros_hbm.at[pl.ds(s * RPT, RPT)],
                    acc_sh.at[pl.ds(s * RPT, RPT)])
    pltpu.sync_copy(ones_hbm, ones_v)
    pltpu.sync_copy(dstr_hbm.at[pl.ds(wid * CPT, CPT)], dst_v)

    plsc.subcore_barrier()

    K8 = 8

    def step(j8, carry):
        base = j8 * K8
        for k in range(K8):
            pltpu.async_copy(ones_v, acc_sh.at[dst_v.at[base + k]], sem,
                             add=True)
        for k in range(K8):
            pltpu.make_async_copy(ones_v, acc_sh.at[dst_v.at[base + k]],
                                  sem).wait()
        return carry

    lax.fori_loop(0, CPT // K8, step, 0)

    plsc.subcore_barrier()
    pltpu.sync_copy(acc_sh.at[pl.ds(s * RPT, RPT)],
                    out_hbm.at[c].at[pl.ds(s * RPT, RPT)])


@functools.partial(
    pl.kernel,
    out_type=jax.ShapeDtypeStruct((NC, NACC, D), jnp.float32),
    mesh=_mesh,
    scratch_types=[
        pltpu.VMEM((SBUF, CHUNK), jnp.int32),
        pltpu.VMEM((DBUF, CHUNK), jnp.int32),
        pltpu.VMEM((CHUNK, D), jnp.float32),
        pltpu.VMEM((CHUNK, D), jnp.float32),
        pltpu.VMEM_SHARED((NACC, D), jnp.float32),
        pltpu.SemaphoreType.DMA,
        pltpu.SemaphoreType.DMA,
        pltpu.SemaphoreType.DMA,
        pltpu.SemaphoreType.DMA,
    ],
)
def _sc_agg(u_hbm, srcr_hbm, dstr_hbm, zeros_hbm, out_hbm,
            src_v, dst_v, gbuf0, gbuf1, acc_sh, sem0, sem1, sem2, sem3):
    c = lax.axis_index("c")
    s = lax.axis_index("s")

    pltpu.sync_copy(zeros_hbm.at[pl.ds(s * RPT, RPT)],
                    acc_sh.at[pl.ds(s * RPT, RPT)])

    def run_core(cpt, base):
        sh = cpt // 2
        dq = cpt // 4
        np2 = cpt // 2

        pltpu.sync_copy(srcr_hbm.at[pl.ds(base, sh)], src_v.at[pl.ds(0, sh)])
        pltpu.sync_copy(dstr_hbm.at[pl.ds(base, dq)], dst_v.at[pl.ds(0, dq)])

        pltpu.async_copy(u_hbm.at[src_v.at[0]], gbuf0, sem0)
        pltpu.async_copy(u_hbm.at[src_v.at[1]], gbuf1, sem1)

        def step(j2, carry):
            c0 = 2 * j2
            seg_d = c0 // dq
            r0 = c0 - seg_d * dq
            rs2 = (c0 + 2) - ((c0 + 2) // sh) * sh
            rs3 = (c0 + 3) - ((c0 + 3) // sh) * sh

            @pl.when((c0 > 0) & (r0 == 0))
            def _():
                pltpu.sync_copy(dstr_hbm.at[pl.ds(base + seg_d * dq, dq)],
                                dst_v.at[pl.ds(0, dq)])

            pltpu.make_async_copy(u_hbm.at[src_v.at[0]], gbuf0, sem0).wait()
            pltpu.async_copy(gbuf0, acc_sh.at[dst_v.at[r0]], sem2, add=True)
            pltpu.make_async_copy(u_hbm.at[src_v.at[0]], gbuf1, sem1).wait()
            pltpu.async_copy(gbuf1, acc_sh.at[dst_v.at[r0 + 1]], sem3,
                             add=True)

            @pl.when(c0 + 2 == sh)
            def _():
                pltpu.sync_copy(srcr_hbm.at[pl.ds(base + sh, sh)],
                                src_v.at[pl.ds(0, sh)])

            @pl.when(j2 + 1 < np2)
            def _():
                pltpu.make_async_copy(gbuf0, acc_sh.at[dst_v.at[r0]],
                                      sem2).wait()
                pltpu.async_copy(u_hbm.at[src_v.at[rs2]], gbuf0, sem0)
                pltpu.make_async_copy(gbuf1, acc_sh.at[dst_v.at[r0 + 1]],
                                      sem3).wait()
                pltpu.async_copy(u_hbm.at[src_v.at[rs3]], gbuf1, sem1)
            return carry

        lax.fori_loop(0, np2, step, 0)

        pltpu.make_async_copy(gbuf0, acc_sh.at[dst_v.at[0]], sem2).wait()
        pltpu.make_async_copy(gbuf1, acc_sh.at[dst_v.at[0]], sem3).wait()

    plsc.subcore_barrier()

    @pl.when(c == 0)
    def _():
        run_core(CPT0, s * CPT0)

    @pl.when(c == 1)
    def _():
        run_core(CPT1, NS * CPT0 + s * CPT1)

    plsc.subcore_barrier()
    pltpu.sync_copy(acc_sh.at[pl.ds(s * RPT, RPT)],
                    out_hbm.at[c].at[pl.ds(s * RPT, RPT)])


BN = 1000
GRID = N // BN


def _mm1a_body(x_ref, w_ref, o_ref):
    o_ref[...] = jnp.dot(x_ref[...], w_ref[...],
                         preferred_element_type=jnp.float32)


_tc_mm1a = pl.pallas_call(
    _mm1a_body,
    grid=(GRID,),
    in_specs=[
        pl.BlockSpec((BN, D), lambda i: (i, 0)),
        pl.BlockSpec((D, D), lambda i: (0, 0)),
    ],
    out_specs=pl.BlockSpec((BN, D), lambda i: (i, 0)),
    out_shape=jax.ShapeDtypeStruct((N, D), jnp.float32),
)


def _mm1b_body(h_ref, dega_ref, degb_ref, o_ref):
    dinv = lax.rsqrt(dega_ref[...] + degb_ref[...] + 1.0)
    o_ref[...] = dinv * h_ref[...]


_tc_mm1b = pl.pallas_call(
    _mm1b_body,
    grid=(GRID,),
    in_specs=[
        pl.BlockSpec((BN, D), lambda i: (i, 0)),
        pl.BlockSpec((BN, 1), lambda i: (i, 0)),
        pl.BlockSpec((BN, 1), lambda i: (i, 0)),
    ],
    out_specs=pl.BlockSpec((BN, D), lambda i: (i, 0)),
    out_shape=jax.ShapeDtypeStruct((N, D), jnp.float32),
)


def _mm2_body(acca_ref, accb_ref, u_ref, dega_ref, degb_ref, b_ref, w_ref,
              o_ref):
    dinv = lax.rsqrt(dega_ref[...] + degb_ref[...] + 1.0)
    z = dinv * (acca_ref[...] + accb_ref[...] + u_ref[...]) + b_ref[...]
    z = jnp.maximum(z, 0.0)
    h = jnp.dot(z, w_ref[...], preferred_element_type=jnp.float32)
    o_ref[...] = dinv * h


_tc_mm2 = pl.pallas_call(
    _mm2_body,
    grid=(GRID,),
    in_specs=[
        pl.BlockSpec((BN, D), lambda i: (i, 0)),
        pl.BlockSpec((BN, D), lambda i: (i, 0)),
        pl.BlockSpec((BN, D), lambda i: (i, 0)),
        pl.BlockSpec((BN, 1), lambda i: (i, 0)),
        pl.BlockSpec((BN, 1), lambda i: (i, 0)),
        pl.BlockSpec((1, D), lambda i: (0, 0)),
        pl.BlockSpec((D, D), lambda i: (0, 0)),
    ],
    out_specs=pl.BlockSpec((BN, D), lambda i: (i, 0)),
    out_shape=jax.ShapeDtypeStruct((N, D), jnp.float32),
)


def _pool_body(acca_ref, accb_ref, u_ref, dega_ref, degb_ref, b_ref,
               batch_ref, o_ref, sums_ref, counts_ref):
    i = pl.program_id(0)

    @pl.when(i == 0)
    def _():
        sums_ref[...] = jnp.zeros_like(sums_ref)
        counts_ref[...] = jnp.zeros_like(counts_ref)

    dinv = lax.rsqrt(dega_ref[...] + degb_ref[...] + 1.0)
    h = dinv * (acca_ref[...] + accb_ref[...] + u_ref[...]) + b_ref[...]
    gids = lax.broadcasted_iota(jnp.int32, (1, G), 1).astype(jnp.float32)
    mask = (batch_ref[...] == gids).astype(jnp.float32)
    sums_ref[...] += lax.dot_general(
        mask, h, (((0,), (0,)), ((), ())),
        preferred_element_type=jnp.float32)
    counts_ref[...] += jnp.sum(mask, axis=0)[:, None]

    @pl.when(i == GRID - 1)
    def _():
        o_ref[...] = sums_ref[...] / jnp.maximum(counts_ref[...], 1.0)


_tc_pool = pl.pallas_call(
    _pool_body,
    grid=(GRID,),
    in_specs=[
        pl.BlockSpec((BN, D), lambda i: (i, 0)),
        pl.BlockSpec((BN, D), lambda i: (i, 0)),
        pl.BlockSpec((BN, D), lambda i: (i, 0)),
        pl.BlockSpec((BN, 1), lambda i: (i, 0)),
        pl.BlockSpec((BN, 1), lambda i: (i, 0)),
        pl.BlockSpec((1, D), lambda i: (0, 0)),
        pl.BlockSpec((BN, 1), lambda i: (i, 0)),
    ],
    out_specs=pl.BlockSpec((G, D), lambda i: (0, 0)),
    out_shape=jax.ShapeDtypeStruct((G, D), jnp.float32),
    scratch_shapes=[
        pltpu.VMEM((G, D), jnp.float32),
        pltpu.VMEM((G, 1), jnp.float32),
    ],
)


def kernel(x, edge_index, batch_index, W1, b1, W2, b2):
    pad_src = jnp.zeros((EPAD,), jnp.int32)
    pad_dst = jnp.full((EPAD,), N, jnp.int32)
    srcr = jnp.concatenate([edge_index[0], pad_src]).reshape(NROWSP, CHUNK)
    dstr = jnp.concatenate([edge_index[1], pad_dst]).reshape(NROWSP, CHUNK)
    batch_col = batch_index.astype(jnp.float32).reshape(N, 1)

    zeros_agg = jnp.zeros((NACC, D), jnp.float32)
    ones_nd = jnp.ones((CHUNK, D), jnp.float32)

    degp = _sc_deg(dstr, ones_nd, zeros_agg)
    dega = degp[0, :N, 0:1]
    degb = degp[1, :N, 0:1]

    h1 = _tc_mm1a(x, W1)
    u1 = _tc_mm1b(h1, dega, degb)
    acc1 = _sc_agg(u1, srcr, dstr, zeros_agg)
    u2 = _tc_mm2(acc1[0, :N], acc1[1, :N], u1, dega, degb,
                 b1.reshape(1, D), W2)
    acc2 = _sc_agg(u2, srcr, dstr, zeros_agg)
    return _tc_pool(acc2[0, :N], acc2[1, :N], u2, dega, degb,
                    b2.reshape(1, D), batch_col)


# speedup vs baseline: 1.0577x; 1.0057x over previous
import functools

import jax
import jax.numpy as jnp
from jax import lax
from jax.experimental import pallas as pl
from jax.experimental.pallas import tpu as pltpu
from jax.experimental.pallas import tpu_sc as plsc

N = 10000
E = 320000
D = 128
G = 16

NC = 2
NS = 16
NW = NC * NS
CHUNK = 128
NROWSP = 2560
EPAD = NROWSP * CHUNK - E
CPT = NROWSP // NW
NACC = 10112
RPT = NACC // NS
HCPT = CPT // 2
CPT0 = 128
CPT1 = 32
SBUF = CPT0 // 2
DBUF = CPT0 // 4

DEG_W = 16
NPADD = 10240
DEG_RPT = NPADD // NS

_mesh = plsc.VectorSubcoreMesh(core_axis_name="c", subcore_axis_name="s")


@functools.partial(
    pl.kernel,
    out_type=jax.ShapeDtypeStruct((NC, NACC, D), jnp.float32),
    mesh=_mesh,
    scratch_types=[
        pltpu.VMEM((CPT, CHUNK), jnp.int32),
        pltpu.VMEM((CHUNK, D), jnp.float32),
        pltpu.VMEM_SHARED((NACC, D), jnp.float32),
        pltpu.SemaphoreType.DMA,
    ],
)
def _sc_deg(dstr_hbm, ones_hbm, zeros_hbm, out_hbm, dst_v, ones_v, acc_sh,
            sem):
    c = lax.axis_index("c")
    s = lax.axis_index("s")
    wid = s * NC + c

    pltpu.sync_copy(zeros_hbm.at[pl.ds(s * RPT, RPT)],
                    acc_sh.at[pl.ds(s * RPT, RPT)])
    pltpu.sync_copy(ones_hbm, ones_v)
    pltpu.sync_copy(dstr_hbm.at[pl.ds(wid * CPT, CPT)], dst_v)

    plsc.subcore_barrier()

    K8 = 8

    def step(j8, carry):
        base = j8 * K8
        for k in range(K8):
            pltpu.async_copy(ones_v, acc_sh.at[dst_v.at[base + k]], sem,
                             add=True)
        for k in range(K8):
            pltpu.make_async_copy(ones_v, acc_sh.at[dst_v.at[base + k]],
                                  sem).wait()
        return carry

    lax.fori_loop(0, CPT // K8, step, 0)

    plsc.subcore_barrier()
    pltpu.sync_copy(acc_sh.at[pl.ds(s * RPT, RPT)],
                    out_hbm.at[c].at[pl.ds(s * RPT, RPT)])


@functools.partial(
    pl.kernel,
    out_type=jax.ShapeDtypeStruct((NC, NACC, D), jnp.float32),
    mesh=_mesh,
    scratch_types=[
        pltpu.VMEM((SBUF, CHUNK), jnp.int32),
        pltpu.VMEM((DBUF, CHUNK), jnp.int32),
        pltpu.VMEM((CHUNK, D), jnp.float32),
        pltpu.VMEM((CHUNK, D), jnp.float32),
        pltpu.VMEM_SHARED((NACC, D), jnp.float32),
        pltpu.SemaphoreType.DMA,
        pltpu.SemaphoreType.DMA,
        pltpu.SemaphoreType.DMA,
        pltpu.SemaphoreType.DMA,
    ],
)
def _sc_agg(u_hbm, srcr_hbm, dstr_hbm, zeros_hbm, out_hbm,
            src_v, dst_v, gbuf0, gbuf1, acc_sh, sem0, sem1, sem2, sem3):
    c = lax.axis_index("c")
    s = lax.axis_index("s")

    pltpu.sync_copy(zeros_hbm.at[pl.ds(s * RPT, RPT)],
                    acc_sh.at[pl.ds(s * RPT, RPT)])

    def run_core(cpt, base):
        sh = cpt // 2
        dq = cpt // 4
        np2 = cpt // 2

        pltpu.sync_copy(srcr_hbm.at[pl.ds(base, sh)], src_v.at[pl.ds(0, sh)])
        pltpu.sync_copy(dstr_hbm.at[pl.ds(base, dq)], dst_v.at[pl.ds(0, dq)])

        pltpu.async_copy(u_hbm.at[src_v.at[0]], gbuf0, sem0)
        pltpu.async_copy(u_hbm.at[src_v.at[1]], gbuf1, sem1)

        def step(j2, carry):
            c0 = 2 * j2
            seg_d = c0 // dq
            r0 = c0 - seg_d * dq
            rs2 = (c0 + 2) - ((c0 + 2) // sh) * sh
            rs3 = (c0 + 3) - ((c0 + 3) // sh) * sh

            @pl.when((c0 > 0) & (r0 == 0))
            def _():
                pltpu.sync_copy(dstr_hbm.at[pl.ds(base + seg_d * dq, dq)],
                                dst_v.at[pl.ds(0, dq)])

            pltpu.make_async_copy(u_hbm.at[src_v.at[0]], gbuf0, sem0).wait()
            pltpu.async_copy(gbuf0, acc_sh.at[dst_v.at[r0]], sem2, add=True)
            pltpu.make_async_copy(u_hbm.at[src_v.at[0]], gbuf1, sem1).wait()
            pltpu.async_copy(gbuf1, acc_sh.at[dst_v.at[r0 + 1]], sem3,
                             add=True)

            @pl.when(c0 + 2 == sh)
            def _():
                pltpu.sync_copy(srcr_hbm.at[pl.ds(base + sh, sh)],
                                src_v.at[pl.ds(0, sh)])

            @pl.when(j2 + 1 < np2)
            def _():
                pltpu.make_async_copy(gbuf0, acc_sh.at[dst_v.at[r0]],
                                      sem2).wait()
                pltpu.async_copy(u_hbm.at[src_v.at[rs2]], gbuf0, sem0)
                pltpu.make_async_copy(gbuf1, acc_sh.at[dst_v.at[r0 + 1]],
                                      sem3).wait()
                pltpu.async_copy(u_hbm.at[src_v.at[rs3]], gbuf1, sem1)
            return carry

        lax.fori_loop(0, np2, step, 0)

        pltpu.make_async_copy(gbuf0, acc_sh.at[dst_v.at[0]], sem2).wait()
        pltpu.make_async_copy(gbuf1, acc_sh.at[dst_v.at[0]], sem3).wait()

    plsc.subcore_barrier()

    @pl.when(c == 0)
    def _():
        run_core(CPT0, s * CPT0)

    @pl.when(c == 1)
    def _():
        run_core(CPT1, NS * CPT0 + s * CPT1)

    plsc.subcore_barrier()
    pltpu.sync_copy(acc_sh.at[pl.ds(s * RPT, RPT)],
                    out_hbm.at[c].at[pl.ds(s * RPT, RPT)])


BN = 1000
GRID = N // BN


def _mm1_body(x_ref, w_ref, dega_ref, degb_ref, o_ref):
    dinv = lax.rsqrt(dega_ref[...] + degb_ref[...] + 1.0)
    h = jnp.dot(x_ref[...], w_ref[...], preferred_element_type=jnp.float32)
    o_ref[...] = dinv * h


_tc_mm1 = pl.pallas_call(
    _mm1_body,
    grid=(GRID,),
    in_specs=[
        pl.BlockSpec((BN, D), lambda i: (i, 0)),
        pl.BlockSpec((D, D), lambda i: (0, 0)),
        pl.BlockSpec((BN, 1), lambda i: (i, 0)),
        pl.BlockSpec((BN, 1), lambda i: (i, 0)),
    ],
    out_specs=pl.BlockSpec((BN, D), lambda i: (i, 0)),
    out_shape=jax.ShapeDtypeStruct((N, D), jnp.float32),
)


def _mm2_body(acca_ref, accb_ref, u_ref, dega_ref, degb_ref, b_ref, w_ref,
              o_ref):
    dinv = lax.rsqrt(dega_ref[...] + degb_ref[...] + 1.0)
    z = dinv * (acca_ref[...] + accb_ref[...] + u_ref[...]) + b_ref[...]
    z = jnp.maximum(z, 0.0)
    h = jnp.dot(z, w_ref[...], preferred_element_type=jnp.float32)
    o_ref[...] = dinv * h


_tc_mm2 = pl.pallas_call(
    _mm2_body,
    grid=(GRID,),
    in_specs=[
        pl.BlockSpec((BN, D), lambda i: (i, 0)),
        pl.BlockSpec((BN, D), lambda i: (i, 0)),
        pl.BlockSpec((BN, D), lambda i: (i, 0)),
        pl.BlockSpec((BN, 1), lambda i: (i, 0)),
        pl.BlockSpec((BN, 1), lambda i: (i, 0)),
        pl.BlockSpec((1, D), lambda i: (0, 0)),
        pl.BlockSpec((D, D), lambda i: (0, 0)),
    ],
    out_specs=pl.BlockSpec((BN, D), lambda i: (i, 0)),
    out_shape=jax.ShapeDtypeStruct((N, D), jnp.float32),
)


def _pool_body(acca_ref, accb_ref, u_ref, dega_ref, degb_ref, b_ref,
               batch_ref, o_ref, sums_ref, counts_ref):
    i = pl.program_id(0)

    @pl.when(i == 0)
    def _():
        sums_ref[...] = jnp.zeros_like(sums_ref)
        counts_ref[...] = jnp.zeros_like(counts_ref)

    dinv = lax.rsqrt(dega_ref[...] + degb_ref[...] + 1.0)
    h = dinv * (acca_ref[...] + accb_ref[...] + u_ref[...]) + b_ref[...]
    gids = lax.broadcasted_iota(jnp.int32, (1, G), 1).astype(jnp.float32)
    mask = (batch_ref[...] == gids).astype(jnp.float32)
    sums_ref[...] += lax.dot_general(
        mask, h, (((0,), (0,)), ((), ())),
        preferred_element_type=jnp.float32)
    counts_ref[...] += jnp.sum(mask, axis=0)[:, None]

    @pl.when(i == GRID - 1)
    def _():
        o_ref[...] = sums_ref[...] / jnp.maximum(counts_ref[...], 1.0)


_tc_pool = pl.pallas_call(
    _pool_body,
    grid=(GRID,),
    in_specs=[
        pl.BlockSpec((BN, D), lambda i: (i, 0)),
        pl.BlockSpec((BN, D), lambda i: (i, 0)),
        pl.BlockSpec((BN, D), lambda i: (i, 0)),
        pl.BlockSpec((BN, 1), lambda i: (i, 0)),
        pl.BlockSpec((BN, 1), lambda i: (i, 0)),
        pl.BlockSpec((1, D), lambda i: (0, 0)),
        pl.BlockSpec((BN, 1), lambda i: (i, 0)),
    ],
    out_specs=pl.BlockSpec((G, D), lambda i: (0, 0)),
    out_shape=jax.ShapeDtypeStruct((G, D), jnp.float32),
    scratch_shapes=[
        pltpu.VMEM((G, D), jnp.float32),
        pltpu.VMEM((G, 1), jnp.float32),
    ],
)


def kernel(x, edge_index, batch_index, W1, b1, W2, b2):
    pad_src = jnp.zeros((EPAD,), jnp.int32)
    pad_dst = jnp.full((EPAD,), N, jnp.int32)
    srcr = jnp.concatenate([edge_index[0], pad_src]).reshape(NROWSP, CHUNK)
    dstr = jnp.concatenate([edge_index[1], pad_dst]).reshape(NROWSP, CHUNK)
    batch_col = batch_index.astype(jnp.float32).reshape(N, 1)

    zeros_agg = jnp.zeros((NACC, D), jnp.float32)
    ones_nd = jnp.ones((CHUNK, D), jnp.float32)

    degp = _sc_deg(dstr, ones_nd, zeros_agg)
    dega = degp[0, :N, 0:1]
    degb = degp[1, :N, 0:1]

    u1 = _tc_mm1(x, W1, dega, degb)
    acc1 = _sc_agg(u1, srcr, dstr, zeros_agg)
    u2 = _tc_mm2(acc1[0, :N], acc1[1, :N], u1, dega, degb,
                 b1.reshape(1, D), W2)
    acc2 = _sc_agg(u2, srcr, dstr, zeros_agg)
    return _tc_pool(acc2[0, :N], acc2[1, :N], u2, dega, degb,
                    b2.reshape(1, D), batch_col)
